# baseline (speedup 1.0000x reference)
import jax
import jax.numpy as jnp
from jax import lax
from jax.experimental import pallas as pl
from jax.experimental.pallas import tpu as pltpu
from jax.experimental.pallas import tpu_sc as plsc

B, L, D = 4, 8192, 1024
T = B * L
NC, NS, LANES = 2, 16, 16
NW = NC * NS
C = 16
NBUF = 6
PF = 2
DJ = D // LANES
GRP = C // LANES

T_SC = 16384
T_TC = T - T_SC
TPW = T_SC // NW
NCHUNK = TPW // C

BT = 512


def _sc_body(x_hbm, ids_hbm, tbl_hbm, out_hbm, xbuf, idbuf, tbl_v, *sems):
    ld_sems, st_sems = sems[:NBUF], sems[NBUF:]
    wid = lax.axis_index("s") * NC + lax.axis_index("c")
    base = wid * TPW

    def start_load(c):
        b = c % NBUF
        tok0 = base + c * C
        return pltpu.async_copy(x_hbm.at[pl.ds(tok0, C), :], xbuf.at[b],
                                ld_sems[b])

    def compute(c):
        b = c % NBUF
        for g in range(GRP):
            idv = idbuf[pl.ds(c * C + g * LANES, LANES)]
            sel = [idv[k] == 1 for k in range(LANES)]

            def j_body(j, carry):
                d0 = j * LANES
                t0 = tbl_v[0, pl.ds(d0, LANES)]
                t1 = tbl_v[1, pl.ds(d0, LANES)]
                for k in range(LANES):
                    row = g * LANES + k
                    emb = jnp.where(sel[k], t1, t0)
                    xbuf[b, row, pl.ds(d0, LANES)] = (
                        xbuf[b, row, pl.ds(d0, LANES)] + emb)
                return carry

            lax.fori_loop(0, DJ, j_body, 0)

    def start_store(c):
        b = c % NBUF
        tok0 = base + c * C
        return pltpu.async_copy(xbuf.at[b], out_hbm.at[pl.ds(tok0, C), :],
                                st_sems[b])

    loads = {}
    stores = {}
    for c in range(min(PF + 1, NCHUNK)):
        loads[c] = start_load(c)
    pltpu.sync_copy(tbl_hbm, tbl_v)
    pltpu.sync_copy(ids_hbm.at[pl.ds(base, TPW)], idbuf)
    for c in range(NCHUNK):
        loads.pop(c).wait()
        nxt = c + 1 + PF
        if nxt < NCHUNK:
            if nxt >= NBUF:
                stores.pop(nxt - NBUF).wait()
            loads[nxt] = start_load(nxt)
        compute(c)
        stores[c] = start_store(c)
    for h in stores.values():
        h.wait()


def _tc_body(x_ref, ids_ref, tbl_ref, out_ref):
    idb = ids_ref[...]
    t0 = tbl_ref[0:1, :]
    t1 = tbl_ref[1:2, :]
    out_ref[...] = x_ref[...] + jnp.where(idb == 1, t1, t0)


def kernel(x, token_type_ids, token_type_table):
    x2 = x.reshape(T, D)
    ids = token_type_ids.reshape(T).astype(jnp.int32)

    sc_fn = pl.kernel(
        _sc_body,
        out_type=jax.ShapeDtypeStruct((T_SC, D), jnp.float32),
        mesh=plsc.VectorSubcoreMesh(
            core_axis_name="c", subcore_axis_name="s",
            num_cores=NC, num_subcores=NS),
        scratch_types=[
            pltpu.VMEM((NBUF, C, D), jnp.float32),
            pltpu.VMEM((TPW,), jnp.int32),
            pltpu.VMEM((2, D), jnp.float32),
        ] + [pltpu.SemaphoreType.DMA] * (2 * NBUF),
    )
    out_sc = sc_fn(x2, ids, token_type_table)

    tsc_blk = T_SC // BT
    tc_fn = pl.pallas_call(
        _tc_body,
        grid=(T_TC // BT,),
        in_specs=[
            pl.BlockSpec((BT, D), lambda i: (tsc_blk + i, 0)),
            pl.BlockSpec((BT, 1), lambda i: (tsc_blk + i, 0)),
            pl.BlockSpec((2, D), lambda i: (0, 0)),
        ],
        out_specs=pl.BlockSpec((BT, D), lambda i: (i, 0)),
        out_shape=jax.ShapeDtypeStruct((T_TC, D), jnp.float32),
    )
    out_tc = tc_fn(x2, ids.reshape(T, 1), token_type_table)

    out = jnp.concatenate([out_sc, out_tc], axis=0)
    return out.reshape(B, L, D)


# speedup vs baseline: 1.8805x; 1.8805x over previous
import jax
import jax.numpy as jnp
from jax import lax
from jax.experimental import pallas as pl
from jax.experimental.pallas import tpu as pltpu
from jax.experimental.pallas import tpu_sc as plsc

B, L, D = 4, 8192, 1024
T = B * L
NC, NS, LANES = 2, 16, 16
NW = NC * NS
C = 16
NBUF = 6
PF = 2
DJ = D // LANES
GRP = C // LANES

T_SC = 0
T_TC = T - T_SC
TPW = T_SC // NW
NCHUNK = TPW // C

BT = 512


def _sc_body(x_hbm, ids_hbm, tbl_hbm, out_hbm, xbuf, idbuf, tbl_v, *sems):
    ld_sems, st_sems = sems[:NBUF], sems[NBUF:]
    wid = lax.axis_index("s") * NC + lax.axis_index("c")
    base = wid * TPW

    def start_load(c):
        b = c % NBUF
        tok0 = base + c * C
        return pltpu.async_copy(x_hbm.at[pl.ds(tok0, C), :], xbuf.at[b],
                                ld_sems[b])

    def compute(c):
        b = c % NBUF
        for g in range(GRP):
            idv = idbuf[pl.ds(c * C + g * LANES, LANES)]
            sel = [idv[k] == 1 for k in range(LANES)]

            def j_body(j, carry):
                d0 = j * LANES
                t0 = tbl_v[0, pl.ds(d0, LANES)]
                t1 = tbl_v[1, pl.ds(d0, LANES)]
                for k in range(LANES):
                    row = g * LANES + k
                    emb = jnp.where(sel[k], t1, t0)
                    xbuf[b, row, pl.ds(d0, LANES)] = (
                        xbuf[b, row, pl.ds(d0, LANES)] + emb)
                return carry

            lax.fori_loop(0, DJ, j_body, 0)

    def start_store(c):
        b = c % NBUF
        tok0 = base + c * C
        return pltpu.async_copy(xbuf.at[b], out_hbm.at[pl.ds(tok0, C), :],
                                st_sems[b])

    loads = {}
    stores = {}
    for c in range(min(PF + 1, NCHUNK)):
        loads[c] = start_load(c)
    pltpu.sync_copy(tbl_hbm, tbl_v)
    pltpu.sync_copy(ids_hbm.at[pl.ds(base, TPW)], idbuf)
    for c in range(NCHUNK):
        loads.pop(c).wait()
        nxt = c + 1 + PF
        if nxt < NCHUNK:
            if nxt >= NBUF:
                stores.pop(nxt - NBUF).wait()
            loads[nxt] = start_load(nxt)
        compute(c)
        stores[c] = start_store(c)
    for h in stores.values():
        h.wait()


def _tc_body(x_ref, ids_ref, tbl_ref, out_ref):
    idb = ids_ref[...]
    t0 = tbl_ref[0:1, :]
    t1 = tbl_ref[1:2, :]
    out_ref[...] = x_ref[...] + jnp.where(idb == 1, t1, t0)


def kernel(x, token_type_ids, token_type_table):
    x2 = x.reshape(T, D)
    ids = token_type_ids.reshape(T).astype(jnp.int32)

    if T_SC == 0:
        tc_fn = pl.pallas_call(
            _tc_body,
            grid=(T // BT,),
            in_specs=[
                pl.BlockSpec((BT, D), lambda i: (i, 0)),
                pl.BlockSpec((BT, 1), lambda i: (i, 0)),
                pl.BlockSpec((2, D), lambda i: (0, 0)),
            ],
            out_specs=pl.BlockSpec((BT, D), lambda i: (i, 0)),
            out_shape=jax.ShapeDtypeStruct((T, D), jnp.float32),
        )
        return tc_fn(x2, ids.reshape(T, 1), token_type_table).reshape(B, L, D)

    sc_fn = pl.kernel(
        _sc_body,
        out_type=jax.ShapeDtypeStruct((T_SC, D), jnp.float32),
        mesh=plsc.VectorSubcoreMesh(
            core_axis_name="c", subcore_axis_name="s",
            num_cores=NC, num_subcores=NS),
        scratch_types=[
            pltpu.VMEM((NBUF, C, D), jnp.float32),
            pltpu.VMEM((TPW,), jnp.int32),
            pltpu.VMEM((2, D), jnp.float32),
        ] + [pltpu.SemaphoreType.DMA] * (2 * NBUF),
    )
    out_sc = sc_fn(x2, ids, token_type_table)

    tsc_blk = T_SC // BT
    tc_fn = pl.pallas_call(
        _tc_body,
        grid=(T_TC // BT,),
        in_specs=[
            pl.BlockSpec((BT, D), lambda i: (tsc_blk + i, 0)),
            pl.BlockSpec((BT, 1), lambda i: (tsc_blk + i, 0)),
            pl.BlockSpec((2, D), lambda i: (0, 0)),
        ],
        out_specs=pl.BlockSpec((BT, D), lambda i: (i, 0)),
        out_shape=jax.ShapeDtypeStruct((T_TC, D), jnp.float32),
    )
    out_tc = tc_fn(x2, ids.reshape(T, 1), token_type_table)

    out = jnp.concatenate([out_sc, out_tc], axis=0)
    return out.reshape(B, L, D)
